# Initial kernel scaffold; baseline (speedup 1.0000x reference)
import jax
import jax.numpy as jnp
from jax.experimental import pallas as pl


def kernel(x, edge_index, W_self1, W_neigh1, b1, W_self2, W_neigh2, b2):
    raise NotImplementedError("write your pallas kernel here")


# speedup vs baseline: 5.1507x; 5.1507x over previous
import functools

import jax
import jax.numpy as jnp
from jax import lax
from jax.experimental import pallas as pl
from jax.experimental.pallas import tpu as pltpu
from jax.experimental.pallas import tpu_sc as plsc

N = 10000
E = 320000
D_IN = 128
D_H = 128
D_OUT = 64

N_PAD = 10240
BN = 2048
GRID = N_PAD // BN
NW = 32
CH = 128
EPW = 10112
E_PAD = NW * EPW
NCH = EPW // CH
RPS = N_PAD // 16


def _seg_body(d, with_deg, *refs):
    if with_deg:
        (p_hbm, src_hbm, dst_hbm, acc_out, deg_out,
         src_v, dst_v, rows_v, ones_v, dz_v, acc_sh, deg_sh, sem) = refs
    else:
        (p_hbm, src_hbm, dst_hbm, acc_out,
         src_v, dst_v, rows_v, acc_sh, sem) = refs
    core = lax.axis_index("c")
    sid = lax.axis_index("s")
    wid = sid * 2 + core

    z16 = jnp.zeros((16,), jnp.float32)
    o16 = jnp.ones((16,), jnp.float32)

    def zrow(i, _):
        rows_v[i // (d // 16), pl.ds((i % (d // 16)) * 16, 16)] = z16
        return _
    lax.fori_loop(0, CH * (d // 16), zrow, None)

    rbase = sid * RPS

    def zcp(t, _):
        pltpu.sync_copy(rows_v, acc_sh.at[pl.ds(rbase + t * CH, CH)])
        return _
    lax.fori_loop(0, RPS // CH, zcp, None)

    if with_deg:
        def zdeg(i, _):
            dz_v[pl.ds(i * 16, 16)] = z16
            return _
        lax.fori_loop(0, RPS // 16, zdeg, None)

        def fill1(i, _):
            ones_v[pl.ds(i * 16, 16)] = o16
            return _
        lax.fori_loop(0, CH // 16, fill1, None)
        pltpu.sync_copy(dz_v, deg_sh.at[pl.ds(rbase, RPS)])

    plsc.subcore_barrier()

    ebase = wid * EPW

    def step(j, _):
        off = ebase + j * CH
        pltpu.sync_copy(src_hbm.at[pl.ds(off, CH)], src_v)
        pltpu.sync_copy(dst_hbm.at[pl.ds(off, CH)], dst_v)
        pltpu.async_copy(p_hbm.at[src_v], rows_v, sem).wait()
        pltpu.sync_copy(rows_v, acc_sh.at[dst_v], add=True)
        if with_deg:
            pltpu.sync_copy(ones_v, deg_sh.at[dst_v], add=True)
        return _
    lax.fori_loop(0, NCH, step, None)

    plsc.subcore_barrier()

    pltpu.sync_copy(acc_sh.at[pl.ds(rbase, RPS)],
                    acc_out.at[core, pl.ds(rbase, RPS)])
    if with_deg:
        pltpu.sync_copy(deg_sh.at[pl.ds(rbase, RPS)],
                        deg_out.at[core, pl.ds(rbase, RPS)])


def _make_segsum(d, with_deg):
    mesh = plsc.VectorSubcoreMesh(core_axis_name="c", subcore_axis_name="s")
    out_type = [jax.ShapeDtypeStruct((2, N_PAD, d), jnp.float32)]
    scratch = [
        pltpu.VMEM((CH,), jnp.int32),
        pltpu.VMEM((CH,), jnp.int32),
        pltpu.VMEM((CH, d), jnp.float32),
    ]
    if with_deg:
        out_type.append(jax.ShapeDtypeStruct((2, N_PAD), jnp.float32))
        scratch += [
            pltpu.VMEM((CH,), jnp.float32),
            pltpu.VMEM((RPS,), jnp.float32),
        ]
    scratch.append(pltpu.VMEM_SHARED((N_PAD, d), jnp.float32))
    if with_deg:
        scratch.append(pltpu.VMEM_SHARED((N_PAD,), jnp.float32))
    scratch.append(pltpu.SemaphoreType.DMA)
    return pl.kernel(
        functools.partial(_seg_body, d, with_deg),
        out_type=tuple(out_type),
        mesh=mesh,
        scratch_types=scratch,
        compiler_params=pltpu.CompilerParams(use_tc_tiling_on_sc=False),
        name=f"sage_segsum_d{d}",
    )


_segsum128 = _make_segsum(D_H, True)
_segsum64 = _make_segsum(D_OUT, False)


def _tc1_body(x_ref, wn_ref, ws_ref, b_ref, p1_ref, xws_ref):
    xb = x_ref[...]
    p1_ref[...] = jnp.dot(xb, wn_ref[...], preferred_element_type=jnp.float32)
    xws_ref[...] = (jnp.dot(xb, ws_ref[...], preferred_element_type=jnp.float32)
                    + b_ref[...])


_tc1 = pl.pallas_call(
    _tc1_body,
    grid=(GRID,),
    in_specs=[
        pl.BlockSpec((BN, D_IN), lambda i: (i, 0)),
        pl.BlockSpec((D_IN, D_H), lambda i: (0, 0)),
        pl.BlockSpec((D_IN, D_H), lambda i: (0, 0)),
        pl.BlockSpec((1, D_H), lambda i: (0, 0)),
    ],
    out_specs=[
        pl.BlockSpec((BN, D_H), lambda i: (i, 0)),
        pl.BlockSpec((BN, D_H), lambda i: (i, 0)),
    ],
    out_shape=[
        jax.ShapeDtypeStruct((N_PAD, D_H), jnp.float32),
        jax.ShapeDtypeStruct((N_PAD, D_H), jnp.float32),
    ],
    name="sage_tc1",
)


def _tc2_body(xws_ref, acc_ref, deg_ref, ws2_ref, wn2_ref, b2_ref,
              p2_ref, hws_ref):
    a = acc_ref[0, :, :] + acc_ref[1, :, :]
    dsum = deg_ref[0, :, :] + deg_ref[1, :, :]
    inv = 1.0 / jnp.maximum(dsum, 1.0)
    h = jnp.maximum(xws_ref[...] + a * inv, 0.0)
    p2_ref[...] = jnp.dot(h, wn2_ref[...], preferred_element_type=jnp.float32)
    hws_ref[...] = (jnp.dot(h, ws2_ref[...], preferred_element_type=jnp.float32)
                    + b2_ref[...])


_tc2 = pl.pallas_call(
    _tc2_body,
    grid=(GRID,),
    in_specs=[
        pl.BlockSpec((BN, D_H), lambda i: (i, 0)),
        pl.BlockSpec((2, BN, D_H), lambda i: (0, i, 0)),
        pl.BlockSpec((2, BN, 1), lambda i: (0, i, 0)),
        pl.BlockSpec((D_H, D_OUT), lambda i: (0, 0)),
        pl.BlockSpec((D_H, D_OUT), lambda i: (0, 0)),
        pl.BlockSpec((1, D_OUT), lambda i: (0, 0)),
    ],
    out_specs=[
        pl.BlockSpec((BN, D_OUT), lambda i: (i, 0)),
        pl.BlockSpec((BN, D_OUT), lambda i: (i, 0)),
    ],
    out_shape=[
        jax.ShapeDtypeStruct((N_PAD, D_OUT), jnp.float32),
        jax.ShapeDtypeStruct((N_PAD, D_OUT), jnp.float32),
    ],
    name="sage_tc2",
)


def _tc3_body(hws_ref, acc_ref, deg_ref, out_ref):
    a = acc_ref[0, :, :] + acc_ref[1, :, :]
    dsum = deg_ref[0, :, :] + deg_ref[1, :, :]
    inv = 1.0 / jnp.maximum(dsum, 1.0)
    out_ref[...] = hws_ref[...] + a * inv


_tc3 = pl.pallas_call(
    _tc3_body,
    grid=(GRID,),
    in_specs=[
        pl.BlockSpec((BN, D_OUT), lambda i: (i, 0)),
        pl.BlockSpec((2, BN, D_OUT), lambda i: (0, i, 0)),
        pl.BlockSpec((2, BN, 1), lambda i: (0, i, 0)),
    ],
    out_specs=pl.BlockSpec((BN, D_OUT), lambda i: (i, 0)),
    out_shape=jax.ShapeDtypeStruct((N_PAD, D_OUT), jnp.float32),
    name="sage_tc3",
)


def kernel(x, edge_index, W_self1, W_neigh1, b1, W_self2, W_neigh2, b2):
    src = edge_index[0]
    dst = edge_index[1]
    pad = E_PAD - E
    src_p = jnp.concatenate([src, jnp.zeros((pad,), jnp.int32)])
    dst_p = jnp.concatenate([dst, jnp.full((pad,), N, jnp.int32)])
    x_p = jnp.pad(x, ((0, N_PAD - N), (0, 0)))

    p1, xws1 = _tc1(x_p, W_neigh1, W_self1, b1.reshape(1, D_H))
    acc1, deg = _segsum128(p1, src_p, dst_p)
    deg3 = deg.reshape(2, N_PAD, 1)
    p2, hws2 = _tc2(xws1, acc1, deg3, W_self2, W_neigh2, b2.reshape(1, D_OUT))
    (acc2,) = _segsum64(p2, src_p, dst_p)
    out = _tc3(hws2, acc2, deg3)
    return out[:N]
